# baseline (speedup 1.0000x reference)
import jax
import jax.numpy as jnp
from jax import lax
from jax.experimental import pallas as pl
from jax.experimental.pallas import tpu as pltpu
from jax.experimental.pallas import tpu_sc as plsc

N_NODES = 100000
D_FEAT = 128
EMB = 32
BATCH = 16384

_NC = 2
_NS = 16
_NW = _NC * _NS
_EV_W = BATCH // _NW
_ROWS_W = 4 * _EV_W
_CHUNK = 128
_NCHUNK = _ROWS_W // _CHUNK

_TBL_W = 128
_COL_DS = 32
_COL_DH = 33

_PROJ_ROWS = 5000
_NGRP = _EV_W // 16


def _proj_body(f_ref, w_ref, b_ref, as_ref, ah_ref, o_ref):
    emb = (
        jnp.dot(f_ref[...], w_ref[...], preferred_element_type=jnp.float32)
        + b_ref[...]
    )
    dot_s = jnp.sum(emb * as_ref[...], axis=1, keepdims=True)
    dot_h = jnp.sum(emb * ah_ref[...], axis=1, keepdims=True)
    pad = jnp.zeros((emb.shape[0], _TBL_W - EMB - 2), jnp.float32)
    o_ref[...] = jnp.concatenate([emb, dot_s, dot_h, pad], axis=1)


def _project(feats, W_fts, b_fts, a):
    return pl.pallas_call(
        _proj_body,
        grid=(N_NODES // _PROJ_ROWS,),
        in_specs=[
            pl.BlockSpec((_PROJ_ROWS, D_FEAT), lambda i: (i, 0)),
            pl.BlockSpec((D_FEAT, EMB), lambda i: (0, 0)),
            pl.BlockSpec((1, EMB), lambda i: (0, 0)),
            pl.BlockSpec((1, EMB), lambda i: (0, 0)),
            pl.BlockSpec((1, EMB), lambda i: (0, 0)),
        ],
        out_specs=pl.BlockSpec((_PROJ_ROWS, _TBL_W), lambda i: (i, 0)),
        out_shape=jax.ShapeDtypeStruct((N_NODES, _TBL_W), jnp.float32),
    )(
        feats, W_fts, b_fts.reshape(1, EMB),
        a[:EMB, 0].reshape(1, EMB), a[EMB:, 0].reshape(1, EMB),
    )


def _fused_body(table_hbm, idx_hbm, scal_hbm, out_hbm,
                idx_v, scal_v, rows_v, out_v, sem):
    wid = lax.axis_index("s") * _NC + lax.axis_index("c")
    pltpu.sync_copy(idx_hbm.at[wid], idx_v)
    pltpu.sync_copy(scal_hbm.at[wid], scal_v)
    iota = jnp.arange(16, dtype=jnp.int32)
    delta = scal_v[pl.ds(5 * _EV_W, 16)]

    def group(g, carry):
        e0 = g * 16
        el = lax.rem(e0, _CHUNK)
        rs = el + iota
        rt = _CHUNK + el + iota
        rh0 = 2 * _CHUNK + el + iota
        rh1 = 3 * _CHUNK + el + iota
        dst = jnp.zeros((16,), jnp.float32)
        dh0 = jnp.zeros((16,), jnp.float32)
        dh1 = jnp.zeros((16,), jnp.float32)
        for c in range(EMB):
            col = jnp.full((16,), c, jnp.int32)
            s_c = plsc.load_gather(rows_v, [rs, col])
            t_c = plsc.load_gather(rows_v, [rt, col])
            h0_c = plsc.load_gather(rows_v, [rh0, col])
            h1_c = plsc.load_gather(rows_v, [rh1, col])
            dst = dst + (s_c - t_c) * (s_c - t_c)
            dh0 = dh0 + (h0_c - t_c) * (h0_c - t_c)
            dh1 = dh1 + (h1_c - t_c) * (h1_c - t_c)
        col_ds = jnp.full((16,), _COL_DS, jnp.int32)
        col_dh = jnp.full((16,), _COL_DH, jnp.int32)
        dots = plsc.load_gather(rows_v, [rs, col_ds])
        dth0 = plsc.load_gather(rows_v, [rh0, col_dh])
        dth1 = plsc.load_gather(rows_v, [rh1, col_dh])
        ev = scal_v[pl.ds(0 * _EV_W + e0, 16)]
        t0 = scal_v[pl.ds(1 * _EV_W + e0, 16)]
        t1 = scal_v[pl.ds(2 * _EV_W + e0, 16)]
        m0 = scal_v[pl.ds(3 * _EV_W + e0, 16)]
        m1 = scal_v[pl.ds(4 * _EV_W + e0, 16)]
        raw0 = dots + dth0
        raw1 = dots + dth1
        d0 = jnp.abs(ev - t0)
        d1 = jnp.abs(ev - t1)
        ep0 = jnp.exp(delta * d0)
        ep1 = jnp.exp(delta * d1)
        en0 = jnp.exp(-delta * d0)
        en1 = jnp.exp(-delta * d1)
        w0 = en0 * raw0
        w1 = en1 * raw1
        sim0 = jnp.where(w0 >= 0, w0, 0.2 * w0)
        sim1 = jnp.where(w1 >= 0, w1, 0.2 * w1)
        mx = jnp.maximum(sim0, sim1)
        a0 = jnp.exp(sim0 - mx)
        a1 = jnp.exp(sim1 - mx)
        inv = 1.0 / (a0 + a1)
        plam = (
            -dst
            - (a0 * inv) * dh0 * ep0 * m0
            - (a1 * inv) * dh1 * ep1 * m1
        )
        out_v[pl.ds(e0, 16)] = plam
        return carry

    for q in range(4):
        copies = []
        for k in range(4):
            j = k * 4 + q
            copies.append(
                pltpu.async_copy(
                    table_hbm.at[idx_v.at[j]],
                    rows_v.at[pl.ds(k * _CHUNK, _CHUNK)],
                    sem,
                )
            )
        for c in copies:
            c.wait()
        lax.fori_loop(8 * q, 8 * (q + 1), group, 0)
    pltpu.sync_copy(out_v, out_hbm.at[pl.ds(wid * _EV_W, _EV_W)])


def _sc_fused(table, idx, scal):
    mesh = plsc.VectorSubcoreMesh(core_axis_name="c", subcore_axis_name="s")
    k = pl.kernel(
        _fused_body,
        out_type=jax.ShapeDtypeStruct((BATCH,), jnp.float32),
        mesh=mesh,
        scratch_types=[
            pltpu.VMEM((_NCHUNK, _CHUNK), jnp.int32),
            pltpu.VMEM((8 * _EV_W,), jnp.float32),
            pltpu.VMEM((4 * _CHUNK, _TBL_W), jnp.float32),
            pltpu.VMEM((_EV_W,), jnp.float32),
            pltpu.SemaphoreType.DMA,
        ],
        compiler_params=pltpu.CompilerParams(needs_layout_passes=False),
    )
    return k(table, idx, scal)


def kernel(feats, W_fts, b_fts, a, delta_s, delta_t,
           s_nodes, t_nodes, event_time,
           s_h_nodes, s_h_times, s_h_time_mask,
           t_h_nodes, t_h_times, t_h_time_mask):
    table = _project(feats, W_fts, b_fts, a)
    idx = (
        jnp.stack([s_nodes, t_nodes, s_h_nodes[:, 0], s_h_nodes[:, 1]], axis=0)
        .astype(jnp.int32)
        .reshape(4, _NW, _EV_W)
        .transpose(1, 0, 2)
        .reshape(_NW, _NCHUNK, _CHUNK)
    )
    zeros = jnp.zeros((_NW, _EV_W), jnp.float32)
    scal = jnp.stack(
        [
            event_time.reshape(_NW, _EV_W),
            s_h_times[:, 0].reshape(_NW, _EV_W),
            s_h_times[:, 1].reshape(_NW, _EV_W),
            s_h_time_mask[:, 0].reshape(_NW, _EV_W),
            s_h_time_mask[:, 1].reshape(_NW, _EV_W),
            jnp.broadcast_to(delta_s.reshape(1, 1), (_NW, _EV_W)),
            zeros,
            zeros,
        ],
        axis=1,
    ).reshape(_NW, 8 * _EV_W)
    return _sc_fused(table, idx, scal)


# speedup vs baseline: 1.0510x; 1.0510x over previous
import jax
import jax.numpy as jnp
from jax import lax
from jax.experimental import pallas as pl
from jax.experimental.pallas import tpu as pltpu
from jax.experimental.pallas import tpu_sc as plsc

N_NODES = 100000
D_FEAT = 128
EMB = 32
BATCH = 16384

_NC = 2
_NS = 16
_NW = _NC * _NS
_EV_W = BATCH // _NW
_SLICES = 8
_EV_S = _EV_W // _SLICES
_ROWS_S = 4 * _EV_S

_TBL_W = 128
_COL_DS = 32
_COL_DH = 33

_PROJ_ROWS = 5000


def _proj_body(f_ref, w_ref, b_ref, as_ref, ah_ref, o_ref):
    emb = (
        jnp.dot(f_ref[...], w_ref[...], preferred_element_type=jnp.float32)
        + b_ref[...]
    )
    dot_s = jnp.sum(emb * as_ref[...], axis=1, keepdims=True)
    dot_h = jnp.sum(emb * ah_ref[...], axis=1, keepdims=True)
    pad = jnp.zeros((emb.shape[0], _TBL_W - EMB - 2), jnp.float32)
    o_ref[...] = jnp.concatenate([emb, dot_s, dot_h, pad], axis=1)


def _project(feats, W_fts, b_fts, a):
    return pl.pallas_call(
        _proj_body,
        grid=(N_NODES // _PROJ_ROWS,),
        in_specs=[
            pl.BlockSpec((_PROJ_ROWS, D_FEAT), lambda i: (i, 0)),
            pl.BlockSpec((D_FEAT, EMB), lambda i: (0, 0)),
            pl.BlockSpec((1, EMB), lambda i: (0, 0)),
            pl.BlockSpec((1, EMB), lambda i: (0, 0)),
            pl.BlockSpec((1, EMB), lambda i: (0, 0)),
        ],
        out_specs=pl.BlockSpec((_PROJ_ROWS, _TBL_W), lambda i: (i, 0)),
        out_shape=jax.ShapeDtypeStruct((N_NODES, _TBL_W), jnp.float32),
    )(
        feats, W_fts, b_fts.reshape(1, EMB),
        a[:EMB, 0].reshape(1, EMB), a[EMB:, 0].reshape(1, EMB),
    )


def _fused_body(table_hbm, idx_hbm, scal_hbm, out_hbm,
                idx_v, scal_v, rows_v, out_v, sem0, sem1):
    wid = lax.axis_index("s") * _NC + lax.axis_index("c")
    pltpu.sync_copy(idx_hbm.at[wid], idx_v)
    pltpu.sync_copy(scal_hbm.at[wid], scal_v)

    iota = jnp.arange(16, dtype=jnp.int32)
    delta = scal_v[pl.ds(5 * _EV_W, 16)]

    def make_group(base):
        def group(g, carry):
            e0 = g * 16
            el = lax.rem(e0, _EV_S)
            rs = base + el + iota
            rt = base + _EV_S + el + iota
            rh0 = base + 2 * _EV_S + el + iota
            rh1 = base + 3 * _EV_S + el + iota
            dst = jnp.zeros((16,), jnp.float32)
            dh0 = jnp.zeros((16,), jnp.float32)
            dh1 = jnp.zeros((16,), jnp.float32)
            for c in range(EMB):
                col = jnp.full((16,), c, jnp.int32)
                s_c = plsc.load_gather(rows_v, [rs, col])
                t_c = plsc.load_gather(rows_v, [rt, col])
                h0_c = plsc.load_gather(rows_v, [rh0, col])
                h1_c = plsc.load_gather(rows_v, [rh1, col])
                dst = dst + (s_c - t_c) * (s_c - t_c)
                dh0 = dh0 + (h0_c - t_c) * (h0_c - t_c)
                dh1 = dh1 + (h1_c - t_c) * (h1_c - t_c)
            col_ds = jnp.full((16,), _COL_DS, jnp.int32)
            col_dh = jnp.full((16,), _COL_DH, jnp.int32)
            dots = plsc.load_gather(rows_v, [rs, col_ds])
            dth0 = plsc.load_gather(rows_v, [rh0, col_dh])
            dth1 = plsc.load_gather(rows_v, [rh1, col_dh])
            ev = scal_v[pl.ds(0 * _EV_W + e0, 16)]
            t0 = scal_v[pl.ds(1 * _EV_W + e0, 16)]
            t1 = scal_v[pl.ds(2 * _EV_W + e0, 16)]
            m0 = scal_v[pl.ds(3 * _EV_W + e0, 16)]
            m1 = scal_v[pl.ds(4 * _EV_W + e0, 16)]
            raw0 = dots + dth0
            raw1 = dots + dth1
            d0 = jnp.abs(ev - t0)
            d1 = jnp.abs(ev - t1)
            ep0 = jnp.exp(delta * d0)
            ep1 = jnp.exp(delta * d1)
            en0 = jnp.exp(-delta * d0)
            en1 = jnp.exp(-delta * d1)
            w0 = en0 * raw0
            w1 = en1 * raw1
            sim0 = jnp.where(w0 >= 0, w0, 0.2 * w0)
            sim1 = jnp.where(w1 >= 0, w1, 0.2 * w1)
            mx = jnp.maximum(sim0, sim1)
            a0 = jnp.exp(sim0 - mx)
            a1 = jnp.exp(sim1 - mx)
            inv = 1.0 / (a0 + a1)
            plam = (
                -dst
                - (a0 * inv) * dh0 * ep0 * m0
                - (a1 * inv) * dh1 * ep1 * m1
            )
            out_v[pl.ds(e0, 16)] = plam
            return carry
        return group

    sems = [sem0, sem1]
    groups = [make_group(0), make_group(_ROWS_S)]

    def issue(s):
        base = (s % 2) * _ROWS_S
        cs = []
        for k in range(4):
            cs.append(
                pltpu.async_copy(
                    table_hbm.at[idx_v.at[k * _SLICES + s]],
                    rows_v.at[pl.ds(base + k * _EV_S, _EV_S)],
                    sems[s % 2],
                )
            )
        return cs

    copies = {0: issue(0)}
    for s in range(_SLICES):
        if s + 1 < _SLICES:
            copies[s + 1] = issue(s + 1)
        for c in copies[s]:
            c.wait()
        lax.fori_loop(4 * s, 4 * (s + 1), groups[s % 2], 0)
    pltpu.sync_copy(out_v, out_hbm.at[pl.ds(wid * _EV_W, _EV_W)])


def _sc_fused(table, idx, scal):
    mesh = plsc.VectorSubcoreMesh(core_axis_name="c", subcore_axis_name="s")
    k = pl.kernel(
        _fused_body,
        out_type=jax.ShapeDtypeStruct((BATCH,), jnp.float32),
        mesh=mesh,
        scratch_types=[
            pltpu.VMEM((4 * _SLICES, _EV_S), jnp.int32),
            pltpu.VMEM((8 * _EV_W,), jnp.float32),
            pltpu.VMEM((2 * _ROWS_S, _TBL_W), jnp.float32),
            pltpu.VMEM((_EV_W,), jnp.float32),
            pltpu.SemaphoreType.DMA,
            pltpu.SemaphoreType.DMA,
        ],
        compiler_params=pltpu.CompilerParams(needs_layout_passes=False),
    )
    return k(table, idx, scal)


def kernel(feats, W_fts, b_fts, a, delta_s, delta_t,
           s_nodes, t_nodes, event_time,
           s_h_nodes, s_h_times, s_h_time_mask,
           t_h_nodes, t_h_times, t_h_time_mask):
    table = _project(feats, W_fts, b_fts, a)
    idx = (
        jnp.stack([s_nodes, t_nodes, s_h_nodes[:, 0], s_h_nodes[:, 1]], axis=0)
        .astype(jnp.int32)
        .reshape(4, _NW, _SLICES, _EV_S)
        .transpose(1, 0, 2, 3)
        .reshape(_NW, 4 * _SLICES, _EV_S)
    )
    zeros = jnp.zeros((_NW, _EV_W), jnp.float32)
    scal = jnp.stack(
        [
            event_time.reshape(_NW, _EV_W),
            s_h_times[:, 0].reshape(_NW, _EV_W),
            s_h_times[:, 1].reshape(_NW, _EV_W),
            s_h_time_mask[:, 0].reshape(_NW, _EV_W),
            s_h_time_mask[:, 1].reshape(_NW, _EV_W),
            jnp.broadcast_to(delta_s.reshape(1, 1), (_NW, _EV_W)),
            zeros,
            zeros,
        ],
        axis=1,
    ).reshape(_NW, 8 * _EV_W)
    return _sc_fused(table, idx, scal)


# speedup vs baseline: 1.1044x; 1.0508x over previous
import jax
import jax.numpy as jnp
from jax import lax
from jax.experimental import pallas as pl
from jax.experimental.pallas import tpu as pltpu
from jax.experimental.pallas import tpu_sc as plsc

N_NODES = 100000
D_FEAT = 128
EMB = 32
BATCH = 16384

_NC = 2
_NS = 16
_NW = _NC * _NS
_EV_W = BATCH // _NW
_SLICES = 8
_EV_S = _EV_W // _SLICES
_ROWS_S = 4 * _EV_S

_TBL_W = 128
_COL_DS = 32
_COL_DH = 33

_PROJ_ROWS = 10000


def _proj_body(f_ref, w_ref, b_ref, as_ref, ah_ref, o_ref):
    emb = (
        jnp.dot(f_ref[...], w_ref[...], preferred_element_type=jnp.float32)
        + b_ref[...]
    )
    dot_s = jnp.sum(emb * as_ref[...], axis=1, keepdims=True)
    dot_h = jnp.sum(emb * ah_ref[...], axis=1, keepdims=True)
    pad = jnp.zeros((emb.shape[0], _TBL_W - EMB - 2), jnp.float32)
    o_ref[...] = jnp.concatenate([emb, dot_s, dot_h, pad], axis=1)


def _project(feats, W_fts, b_fts, a):
    return pl.pallas_call(
        _proj_body,
        grid=(N_NODES // _PROJ_ROWS,),
        in_specs=[
            pl.BlockSpec((_PROJ_ROWS, D_FEAT), lambda i: (i, 0)),
            pl.BlockSpec((D_FEAT, EMB), lambda i: (0, 0)),
            pl.BlockSpec((1, EMB), lambda i: (0, 0)),
            pl.BlockSpec((1, EMB), lambda i: (0, 0)),
            pl.BlockSpec((1, EMB), lambda i: (0, 0)),
        ],
        out_specs=pl.BlockSpec((_PROJ_ROWS, _TBL_W), lambda i: (i, 0)),
        out_shape=jax.ShapeDtypeStruct((N_NODES, _TBL_W), jnp.float32),
    )(
        feats, W_fts, b_fts.reshape(1, EMB),
        a[:EMB, 0].reshape(1, EMB), a[EMB:, 0].reshape(1, EMB),
    )


def _fused_body(table_hbm, idx_hbm, scal_hbm, out_hbm,
                idx_v, scal_v, rows_v, out_v, sem0, sem1):
    wid = lax.axis_index("s") * _NC + lax.axis_index("c")
    pltpu.sync_copy(idx_hbm.at[wid], idx_v)
    pltpu.sync_copy(scal_hbm.at[wid], scal_v)

    iota = jnp.arange(16, dtype=jnp.int32)
    delta = scal_v[pl.ds(5 * _EV_W, 16)]

    def make_group(base):
        def group(g, carry):
            e0 = g * 16
            el = lax.rem(e0, _EV_S)
            rs = base + el + iota
            rt = base + _EV_S + el + iota
            rh0 = base + 2 * _EV_S + el + iota
            rh1 = base + 3 * _EV_S + el + iota
            dst = jnp.zeros((16,), jnp.float32)
            dh0 = jnp.zeros((16,), jnp.float32)
            dh1 = jnp.zeros((16,), jnp.float32)
            for c in range(EMB):
                col = jnp.full((16,), c, jnp.int32)
                s_c = plsc.load_gather(rows_v, [rs, col])
                t_c = plsc.load_gather(rows_v, [rt, col])
                h0_c = plsc.load_gather(rows_v, [rh0, col])
                h1_c = plsc.load_gather(rows_v, [rh1, col])
                dst = dst + (s_c - t_c) * (s_c - t_c)
                dh0 = dh0 + (h0_c - t_c) * (h0_c - t_c)
                dh1 = dh1 + (h1_c - t_c) * (h1_c - t_c)
            col_ds = jnp.full((16,), _COL_DS, jnp.int32)
            col_dh = jnp.full((16,), _COL_DH, jnp.int32)
            dots = plsc.load_gather(rows_v, [rs, col_ds])
            dth0 = plsc.load_gather(rows_v, [rh0, col_dh])
            dth1 = plsc.load_gather(rows_v, [rh1, col_dh])
            ev = scal_v[pl.ds(0 * _EV_W + e0, 16)]
            t0 = scal_v[pl.ds(1 * _EV_W + e0, 16)]
            t1 = scal_v[pl.ds(2 * _EV_W + e0, 16)]
            m0 = scal_v[pl.ds(3 * _EV_W + e0, 16)]
            m1 = scal_v[pl.ds(4 * _EV_W + e0, 16)]
            raw0 = dots + dth0
            raw1 = dots + dth1
            d0 = jnp.abs(ev - t0)
            d1 = jnp.abs(ev - t1)
            ep0 = jnp.exp(delta * d0)
            ep1 = jnp.exp(delta * d1)
            en0 = jnp.exp(-delta * d0)
            en1 = jnp.exp(-delta * d1)
            w0 = en0 * raw0
            w1 = en1 * raw1
            sim0 = jnp.where(w0 >= 0, w0, 0.2 * w0)
            sim1 = jnp.where(w1 >= 0, w1, 0.2 * w1)
            mx = jnp.maximum(sim0, sim1)
            a0 = jnp.exp(sim0 - mx)
            a1 = jnp.exp(sim1 - mx)
            inv = 1.0 / (a0 + a1)
            plam = (
                -dst
                - (a0 * inv) * dh0 * ep0 * m0
                - (a1 * inv) * dh1 * ep1 * m1
            )
            out_v[pl.ds(e0, 16)] = plam
            return carry
        return group

    sems = [sem0, sem1]
    groups = [make_group(0), make_group(_ROWS_S)]

    def issue(s):
        base = (s % 2) * _ROWS_S
        cs = []
        for k in range(4):
            cs.append(
                pltpu.async_copy(
                    table_hbm.at[idx_v.at[k * _SLICES + s]],
                    rows_v.at[pl.ds(base + k * _EV_S, _EV_S)],
                    sems[s % 2],
                )
            )
        return cs

    copies = {0: issue(0)}
    for s in range(_SLICES):
        if s + 1 < _SLICES:
            copies[s + 1] = issue(s + 1)
        for c in copies[s]:
            c.wait()
        lax.fori_loop(4 * s, 4 * (s + 1), groups[s % 2], 0)
    pltpu.sync_copy(out_v, out_hbm.at[pl.ds(wid * _EV_W, _EV_W)])


def _sc_fused(table, idx, scal):
    mesh = plsc.VectorSubcoreMesh(core_axis_name="c", subcore_axis_name="s")
    k = pl.kernel(
        _fused_body,
        out_type=jax.ShapeDtypeStruct((BATCH,), jnp.float32),
        mesh=mesh,
        scratch_types=[
            pltpu.VMEM((4 * _SLICES, _EV_S), jnp.int32),
            pltpu.VMEM((8 * _EV_W,), jnp.float32),
            pltpu.VMEM((2 * _ROWS_S, _TBL_W), jnp.float32),
            pltpu.VMEM((_EV_W,), jnp.float32),
            pltpu.SemaphoreType.DMA,
            pltpu.SemaphoreType.DMA,
        ],
        compiler_params=pltpu.CompilerParams(needs_layout_passes=False),
    )
    return k(table, idx, scal)


def kernel(feats, W_fts, b_fts, a, delta_s, delta_t,
           s_nodes, t_nodes, event_time,
           s_h_nodes, s_h_times, s_h_time_mask,
           t_h_nodes, t_h_times, t_h_time_mask):
    table = _project(feats, W_fts, b_fts, a)
    idx = (
        jnp.stack([s_nodes, t_nodes, s_h_nodes[:, 0], s_h_nodes[:, 1]], axis=0)
        .astype(jnp.int32)
        .reshape(4, _NW, _SLICES, _EV_S)
        .transpose(1, 0, 2, 3)
        .reshape(_NW, 4 * _SLICES, _EV_S)
    )
    zeros = jnp.zeros((_NW, _EV_W), jnp.float32)
    scal = jnp.stack(
        [
            event_time.reshape(_NW, _EV_W),
            s_h_times[:, 0].reshape(_NW, _EV_W),
            s_h_times[:, 1].reshape(_NW, _EV_W),
            s_h_time_mask[:, 0].reshape(_NW, _EV_W),
            s_h_time_mask[:, 1].reshape(_NW, _EV_W),
            jnp.broadcast_to(delta_s.reshape(1, 1), (_NW, _EV_W)),
            zeros,
            zeros,
        ],
        axis=1,
    ).reshape(_NW, 8 * _EV_W)
    return _sc_fused(table, idx, scal)
